# baseline (speedup 1.0000x reference)
import math

import jax
import jax.numpy as jnp
from jax.experimental import pallas as pl
from jax.experimental.pallas import tpu as pltpu

N_ROWS = 200000
PACK = 4
NR = N_ROWS // PACK
BR = 2000
NB = NR // BR
K_IN = 32 * PACK
H_P = 64 * PACK
O_P = 4 * PACK


def _stats_kernel(feat_ref, w1_ref, s1_ref, s2_ref):
    i = pl.program_id(0)
    x = feat_ref[...].astype(jnp.bfloat16)
    h = jnp.dot(x, w1_ref[...], preferred_element_type=jnp.float32)
    s1 = jnp.sum(h, axis=0, keepdims=True)
    s2 = jnp.sum(h * h, axis=0, keepdims=True)

    @pl.when(i == 0)
    def _init():
        s1_ref[...] = s1
        s2_ref[...] = s2

    @pl.when(i != 0)
    def _acc():
        s1_ref[...] += s1
        s2_ref[...] += s2


def _apply_kernel(feat_ref, w1_ref, scale_ref, shift_ref, w2_ref, b2_ref,
                  out_ref):
    x = feat_ref[...].astype(jnp.bfloat16)
    h = jnp.dot(x, w1_ref[...], preferred_element_type=jnp.float32)
    hn = h * scale_ref[...] + shift_ref[...]
    g = hn * 0.5 * (1.0 + jax.lax.erf(hn * (1.0 / math.sqrt(2.0))))
    o = jnp.dot(g.astype(jnp.bfloat16), w2_ref[...],
                preferred_element_type=jnp.float32)
    out_ref[...] = jax.nn.sigmoid(o + b2_ref[...])


def kernel(feat, W1, b1, gamma, beta, W2, b2):
    del b1
    featr = feat.reshape(NR, K_IN)
    eye4 = jnp.eye(PACK, dtype=jnp.float32)
    w1e = jnp.kron(eye4, W1.T).astype(jnp.bfloat16)
    w2e = jnp.kron(eye4, W2.T).astype(jnp.bfloat16)

    s1, s2 = pl.pallas_call(
        _stats_kernel,
        grid=(NB,),
        in_specs=[
            pl.BlockSpec((BR, K_IN), lambda i: (i, 0)),
            pl.BlockSpec((K_IN, H_P), lambda i: (0, 0)),
        ],
        out_specs=[
            pl.BlockSpec((1, H_P), lambda i: (0, 0)),
            pl.BlockSpec((1, H_P), lambda i: (0, 0)),
        ],
        out_shape=[
            jax.ShapeDtypeStruct((1, H_P), jnp.float32),
            jax.ShapeDtypeStruct((1, H_P), jnp.float32),
        ],
        compiler_params=pltpu.CompilerParams(
            dimension_semantics=("arbitrary",)),
    )(featr, w1e)

    s1 = s1.reshape(PACK, 64).sum(axis=0)
    s2 = s2.reshape(PACK, 64).sum(axis=0)
    mean = s1 / N_ROWS
    var = s2 / N_ROWS - mean * mean
    scale = gamma * jax.lax.rsqrt(var + 1e-5)
    shift = beta - mean * scale
    scale4 = jnp.tile(scale, PACK).reshape(1, H_P)
    shift4 = jnp.tile(shift, PACK).reshape(1, H_P)
    b2e = jnp.tile(b2, PACK).reshape(1, O_P)

    out = pl.pallas_call(
        _apply_kernel,
        grid=(NB,),
        in_specs=[
            pl.BlockSpec((BR, K_IN), lambda i: (i, 0)),
            pl.BlockSpec((K_IN, H_P), lambda i: (0, 0)),
            pl.BlockSpec((1, H_P), lambda i: (0, 0)),
            pl.BlockSpec((1, H_P), lambda i: (0, 0)),
            pl.BlockSpec((H_P, O_P), lambda i: (0, 0)),
            pl.BlockSpec((1, O_P), lambda i: (0, 0)),
        ],
        out_specs=pl.BlockSpec((BR, O_P), lambda i: (i, 0)),
        out_shape=jax.ShapeDtypeStruct((NR, O_P), jnp.float32),
        compiler_params=pltpu.CompilerParams(
            dimension_semantics=("arbitrary",)),
    )(featr, w1e, scale4, shift4, w2e, b2e)

    return out.reshape(N_ROWS, 4)


# speedup vs baseline: 1.2821x; 1.2821x over previous
import math

import jax
import jax.numpy as jnp
from jax.experimental import pallas as pl
from jax.experimental.pallas import tpu as pltpu

N_ROWS = 200000
BR = 8000
NB = N_ROWS // BR
IN_CH = 32
HID = 64
OUT = 4


def _stats_kernel(feat_ref, w1_ref, s1_ref, s2_ref):
    i = pl.program_id(0)
    x = feat_ref[...].astype(jnp.bfloat16)
    h = jnp.dot(x, w1_ref[...], preferred_element_type=jnp.float32)
    s1 = jnp.sum(h, axis=0, keepdims=True)
    s2 = jnp.sum(h * h, axis=0, keepdims=True)

    @pl.when(i == 0)
    def _init():
        s1_ref[...] = s1
        s2_ref[...] = s2

    @pl.when(i != 0)
    def _acc():
        s1_ref[...] += s1
        s2_ref[...] += s2


def _apply_kernel(feat_ref, w1_ref, scale_ref, shift_ref, w2_ref, b2_ref,
                  out_ref):
    x = feat_ref[...].astype(jnp.bfloat16)
    h = jnp.dot(x, w1_ref[...], preferred_element_type=jnp.float32)
    hn = h * scale_ref[...] + shift_ref[...]
    g = hn * 0.5 * (1.0 + jax.lax.erf(hn * (1.0 / math.sqrt(2.0))))
    o = jnp.dot(g.astype(jnp.bfloat16), w2_ref[...],
                preferred_element_type=jnp.float32)
    out_ref[...] = jax.nn.sigmoid(o + b2_ref[...])


def kernel(feat, W1, b1, gamma, beta, W2, b2):
    del b1
    w1t = W1.T.astype(jnp.bfloat16)
    w2t = W2.T.astype(jnp.bfloat16)

    s1, s2 = pl.pallas_call(
        _stats_kernel,
        grid=(NB,),
        in_specs=[
            pl.BlockSpec((BR, IN_CH), lambda i: (i, 0)),
            pl.BlockSpec((IN_CH, HID), lambda i: (0, 0)),
        ],
        out_specs=[
            pl.BlockSpec((1, HID), lambda i: (0, 0)),
            pl.BlockSpec((1, HID), lambda i: (0, 0)),
        ],
        out_shape=[
            jax.ShapeDtypeStruct((1, HID), jnp.float32),
            jax.ShapeDtypeStruct((1, HID), jnp.float32),
        ],
        compiler_params=pltpu.CompilerParams(
            dimension_semantics=("arbitrary",)),
    )(feat, w1t)

    mean = s1 / N_ROWS
    var = s2 / N_ROWS - mean * mean
    scale = gamma * jax.lax.rsqrt(var + 1e-5)
    shift = beta - mean * scale
    b2r = b2.reshape(1, OUT)

    out = pl.pallas_call(
        _apply_kernel,
        grid=(NB,),
        in_specs=[
            pl.BlockSpec((BR, IN_CH), lambda i: (i, 0)),
            pl.BlockSpec((IN_CH, HID), lambda i: (0, 0)),
            pl.BlockSpec((1, HID), lambda i: (0, 0)),
            pl.BlockSpec((1, HID), lambda i: (0, 0)),
            pl.BlockSpec((HID, OUT), lambda i: (0, 0)),
            pl.BlockSpec((1, OUT), lambda i: (0, 0)),
        ],
        out_specs=pl.BlockSpec((BR, OUT), lambda i: (i, 0)),
        out_shape=jax.ShapeDtypeStruct((N_ROWS, OUT), jnp.float32),
        compiler_params=pltpu.CompilerParams(
            dimension_semantics=("arbitrary",)),
    )(feat, w1t, scale, shift, w2t, b2r)

    return out


# speedup vs baseline: 3.6369x; 2.8367x over previous
import math

import jax
import jax.numpy as jnp
from jax.experimental import pallas as pl
from jax.experimental.pallas import tpu as pltpu

N_ROWS = 200000
BL = 8192
NBL = -(-N_ROWS // BL)
IN_CH = 32
HID = 64
OUT = 4
LT = 128


def _stats_kernel(feat_ref, w1_ref, s1_ref, s2_ref):
    i = pl.program_id(0)
    x = feat_ref[...]
    lane = jax.lax.broadcasted_iota(jnp.int32, (IN_CH, BL), 1)
    x = jnp.where(lane < (N_ROWS - i * BL), x, 0.0)
    h = jnp.dot(w1_ref[...], x.astype(jnp.bfloat16),
                preferred_element_type=jnp.float32)
    h3 = h.reshape(HID, BL // LT, LT)
    s1 = h3.sum(axis=1)
    s2 = (h3 * h3).sum(axis=1)

    @pl.when(i == 0)
    def _init():
        s1_ref[...] = s1
        s2_ref[...] = s2

    @pl.when(i != 0)
    def _acc():
        s1_ref[...] += s1
        s2_ref[...] += s2


def _apply_kernel(feat_ref, w1_ref, scale_ref, shift_ref, w2_ref, b2_ref,
                  out_ref):
    x = feat_ref[...].astype(jnp.bfloat16)
    h = jnp.dot(w1_ref[...], x,
                preferred_element_type=jnp.float32)
    h3 = h.reshape(HID, BL // LT, LT)
    hn = h3 * scale_ref[...][:, None, :] + shift_ref[...][:, None, :]
    g = hn * 0.5 * (1.0 + jax.lax.erf(hn * (1.0 / math.sqrt(2.0))))
    o = jnp.dot(w2_ref[...], g.reshape(HID, BL).astype(jnp.bfloat16),
                preferred_element_type=jnp.float32)
    o3 = o.reshape(OUT, BL // LT, LT) + b2_ref[...][:, None, :]
    out_ref[...] = jax.nn.sigmoid(o3).reshape(OUT, BL)


def kernel(feat, W1, b1, gamma, beta, W2, b2):
    del b1
    featT = feat.T
    w1b = W1.astype(jnp.bfloat16)
    w2b = W2.astype(jnp.bfloat16)

    s1t, s2t = pl.pallas_call(
        _stats_kernel,
        grid=(NBL,),
        in_specs=[
            pl.BlockSpec((IN_CH, BL), lambda i: (0, i)),
            pl.BlockSpec((HID, IN_CH), lambda i: (0, 0)),
        ],
        out_specs=[
            pl.BlockSpec((HID, LT), lambda i: (0, 0)),
            pl.BlockSpec((HID, LT), lambda i: (0, 0)),
        ],
        out_shape=[
            jax.ShapeDtypeStruct((HID, LT), jnp.float32),
            jax.ShapeDtypeStruct((HID, LT), jnp.float32),
        ],
        compiler_params=pltpu.CompilerParams(
            dimension_semantics=("arbitrary",)),
    )(featT, w1b)

    mean = s1t.sum(axis=1) / N_ROWS
    var = s2t.sum(axis=1) / N_ROWS - mean * mean
    scale = gamma * jax.lax.rsqrt(var + 1e-5)
    shift = beta - mean * scale
    scale128 = jnp.broadcast_to(scale[:, None], (HID, LT))
    shift128 = jnp.broadcast_to(shift[:, None], (HID, LT))
    b2_128 = jnp.broadcast_to(b2[:, None], (OUT, LT))

    outT = pl.pallas_call(
        _apply_kernel,
        grid=(NBL,),
        in_specs=[
            pl.BlockSpec((IN_CH, BL), lambda i: (0, i)),
            pl.BlockSpec((HID, IN_CH), lambda i: (0, 0)),
            pl.BlockSpec((HID, LT), lambda i: (0, 0)),
            pl.BlockSpec((HID, LT), lambda i: (0, 0)),
            pl.BlockSpec((OUT, HID), lambda i: (0, 0)),
            pl.BlockSpec((OUT, LT), lambda i: (0, 0)),
        ],
        out_specs=pl.BlockSpec((OUT, BL), lambda i: (0, i)),
        out_shape=jax.ShapeDtypeStruct((OUT, N_ROWS), jnp.float32),
        compiler_params=pltpu.CompilerParams(
            dimension_semantics=("arbitrary",)),
    )(featT, w1b, scale128, shift128, w2b, b2_128)

    return outT.T


# speedup vs baseline: 4.8066x; 1.3216x over previous
import math

import jax
import jax.numpy as jnp
from jax.experimental import pallas as pl
from jax.experimental.pallas import tpu as pltpu

N_ROWS = 200000
BL = 8192
NBL = -(-N_ROWS // BL)
IN_CH = 32
HID = 64
OUT = 4
MCOL = 8


def _stats_kernel(x_ref, ones_ref, g_ref, m_ref):
    i = pl.program_id(0)
    x = x_ref[...]
    lane = jax.lax.broadcasted_iota(jnp.int32, (IN_CH, BL), 1)
    x = jnp.where(lane < (N_ROWS - i * BL), x, 0.0)
    xb = x.astype(jnp.bfloat16)
    g = jax.lax.dot_general(xb, xb, (((1,), (1,)), ((), ())),
                            preferred_element_type=jnp.float32)
    m = jnp.dot(xb, ones_ref[...],
                preferred_element_type=jnp.float32)

    @pl.when(i == 0)
    def _init():
        g_ref[...] = g
        m_ref[...] = m

    @pl.when(i != 0)
    def _acc():
        g_ref[...] += g
        m_ref[...] += m


def _apply_kernel(x_ref, w1s_ref, shift_ref, w2_ref, b2_ref, out_ref):
    xb = x_ref[...].astype(jnp.bfloat16)
    h = jnp.dot(w1s_ref[...], xb,
                preferred_element_type=jnp.float32)
    hn = h + jnp.broadcast_to(shift_ref[...], (HID, BL))
    u = 0.5 * (1.0 + jax.lax.erf(hn * (1.0 / math.sqrt(2.0))))
    g = hn * u
    o = jnp.dot(w2_ref[...], g.astype(jnp.bfloat16),
                preferred_element_type=jnp.float32)
    ob = o + jnp.broadcast_to(b2_ref[...], (OUT, BL))
    out_ref[...] = jax.nn.sigmoid(ob)


def kernel(feat, W1, b1, gamma, beta, W2, b2):
    del b1
    featT = feat.T
    ones_col = jnp.ones((BL, MCOL), dtype=jnp.bfloat16)

    g_acc, m_acc = pl.pallas_call(
        _stats_kernel,
        grid=(NBL,),
        in_specs=[
            pl.BlockSpec((IN_CH, BL), lambda i: (0, i)),
            pl.BlockSpec((BL, MCOL), lambda i: (0, 0)),
        ],
        out_specs=[
            pl.BlockSpec((IN_CH, IN_CH), lambda i: (0, 0)),
            pl.BlockSpec((IN_CH, MCOL), lambda i: (0, 0)),
        ],
        out_shape=[
            jax.ShapeDtypeStruct((IN_CH, IN_CH), jnp.float32),
            jax.ShapeDtypeStruct((IN_CH, MCOL), jnp.float32),
        ],
        compiler_params=pltpu.CompilerParams(
            dimension_semantics=("arbitrary",)),
    )(featT, ones_col)

    mu = m_acc[:, 0] / N_ROWS
    mean = W1 @ mu
    t = W1 @ (g_acc / N_ROWS)
    ex2 = jnp.sum(t * W1, axis=1)
    var = ex2 - mean * mean
    scale = gamma * jax.lax.rsqrt(var + 1e-5)
    shift = beta - mean * scale
    w1s = (W1 * scale[:, None]).astype(jnp.bfloat16)
    w2b = W2.astype(jnp.bfloat16)

    outT = pl.pallas_call(
        _apply_kernel,
        grid=(NBL,),
        in_specs=[
            pl.BlockSpec((IN_CH, BL), lambda i: (0, i)),
            pl.BlockSpec((HID, IN_CH), lambda i: (0, 0)),
            pl.BlockSpec((HID, 1), lambda i: (0, 0)),
            pl.BlockSpec((OUT, HID), lambda i: (0, 0)),
            pl.BlockSpec((OUT, 1), lambda i: (0, 0)),
        ],
        out_specs=pl.BlockSpec((OUT, BL), lambda i: (0, i)),
        out_shape=jax.ShapeDtypeStruct((OUT, N_ROWS), jnp.float32),
        compiler_params=pltpu.CompilerParams(
            dimension_semantics=("arbitrary",)),
    )(featT, w1s, shift[:, None], w2b, b2[:, None])

    return outT.T


# speedup vs baseline: 5.6789x; 1.1815x over previous
import math

import jax
import jax.numpy as jnp
from jax.experimental import pallas as pl
from jax.experimental.pallas import tpu as pltpu

N_ROWS = 200000
BL = 8192
NBL = -(-N_ROWS // BL)
IN_CH = 32
HID = 64
OUT = 4
MROW = 8

_CONTRACT_0_0 = (((0,), (0,)), ((), ()))
_CONTRACT_1_1 = (((1,), (1,)), ((), ()))


def _fused_kernel(x_ref, w1t_ref, gamma_ref, beta_ref, w2t_ref, b2_ref,
                  out_ref, g_ref, m_ref, w1s_ref, sh_ref, b2c_ref):
    p = pl.program_id(0)
    i = pl.program_id(1)

    @pl.when(p == 0)
    def _stats():
        x = x_ref[...]
        lane = jax.lax.broadcasted_iota(jnp.int32, (IN_CH, BL), 1)
        x = jnp.where(lane < (N_ROWS - i * BL), x, 0.0)
        xb = x.astype(jnp.bfloat16)
        g = jax.lax.dot_general(xb, xb, _CONTRACT_1_1,
                                preferred_element_type=jnp.float32)
        ones = jnp.ones((MROW, BL), dtype=jnp.bfloat16)
        m = jax.lax.dot_general(ones, xb, _CONTRACT_1_1,
                                preferred_element_type=jnp.float32)

        @pl.when(i == 0)
        def _init():
            g_ref[...] = g
            m_ref[...] = m

        @pl.when(i != 0)
        def _acc():
            g_ref[...] += g
            m_ref[...] += m

    @pl.when((p == 1) & (i == 0))
    def _finalize():
        wt = w1t_ref[...]
        mu = m_ref[0:1, :] / N_ROWS
        mean = jnp.dot(mu, wt,
                       preferred_element_type=jnp.float32)
        t = jnp.dot(g_ref[...] / N_ROWS, wt,
                    preferred_element_type=jnp.float32)
        ex2 = jnp.sum(t * wt, axis=0, keepdims=True)
        var = ex2 - mean * mean
        scale = gamma_ref[...] * jax.lax.rsqrt(var + 1e-5)
        shift = beta_ref[...] - mean * scale
        w1s_ref[...] = (wt * scale).astype(jnp.bfloat16)
        sh_ref[...] = shift.T
        b2c_ref[...] = b2_ref[...].T

    @pl.when(p == 1)
    def _apply():
        xb = x_ref[...].astype(jnp.bfloat16)
        h = jax.lax.dot_general(w1s_ref[...], xb, _CONTRACT_0_0,
                                preferred_element_type=jnp.float32)
        hn = h + jnp.broadcast_to(sh_ref[...], (HID, BL))
        u = 0.5 * (1.0 + jax.lax.erf(hn * (1.0 / math.sqrt(2.0))))
        gl = hn * u
        o = jnp.dot(w2t_ref[...].astype(jnp.bfloat16),
                    gl.astype(jnp.bfloat16),
                    preferred_element_type=jnp.float32)
        ob = o + jnp.broadcast_to(b2c_ref[...], (OUT, BL))
        out_ref[...] = jax.nn.sigmoid(ob)


def kernel(feat, W1, b1, gamma, beta, W2, b2):
    del b1
    outT = pl.pallas_call(
        _fused_kernel,
        grid=(2, NBL),
        in_specs=[
            pl.BlockSpec((IN_CH, BL), lambda p, i: (0, i)),
            pl.BlockSpec((IN_CH, HID), lambda p, i: (0, 0)),
            pl.BlockSpec((1, HID), lambda p, i: (0, 0)),
            pl.BlockSpec((1, HID), lambda p, i: (0, 0)),
            pl.BlockSpec((OUT, HID), lambda p, i: (0, 0)),
            pl.BlockSpec((1, OUT), lambda p, i: (0, 0)),
        ],
        out_specs=pl.BlockSpec((OUT, BL), lambda p, i: (0, i)),
        out_shape=jax.ShapeDtypeStruct((OUT, N_ROWS), jnp.float32),
        scratch_shapes=[
            pltpu.VMEM((IN_CH, IN_CH), jnp.float32),
            pltpu.VMEM((MROW, IN_CH), jnp.float32),
            pltpu.VMEM((IN_CH, HID), jnp.bfloat16),
            pltpu.VMEM((HID, 1), jnp.float32),
            pltpu.VMEM((OUT, 1), jnp.float32),
        ],
        compiler_params=pltpu.CompilerParams(
            dimension_semantics=("arbitrary", "arbitrary")),
    )(feat.T, W1.T, gamma[None, :], beta[None, :], W2, b2[None, :])

    return outT.T


# speedup vs baseline: 7.9146x; 1.3937x over previous
import math

import jax
import jax.numpy as jnp
from jax.experimental import pallas as pl
from jax.experimental.pallas import tpu as pltpu

N_ROWS = 200000
BL = 16384
NBL = -(-N_ROWS // BL)
IN_CH = 32
HID = 64
OUT = 4
MROW = 8

_CONTRACT_0_0 = (((0,), (0,)), ((), ()))
_CONTRACT_1_1 = (((1,), (1,)), ((), ()))


def _fused_kernel(x_ref, w1t_ref, gamma_ref, beta_ref, w2t_ref, b2_ref,
                  out_ref, g_ref, m_ref, w1s_ref, sh_ref, b2c_ref, w2h_ref):
    p = pl.program_id(0)
    i = pl.program_id(1)

    @pl.when(p == 0)
    def _stats():
        x = x_ref[...]
        lane = jax.lax.broadcasted_iota(jnp.int32, (IN_CH, BL), 1)
        x = jnp.where(lane < (N_ROWS - i * BL), x, 0.0)
        xb = x.astype(jnp.bfloat16)
        g = jax.lax.dot_general(xb, xb, _CONTRACT_1_1,
                                preferred_element_type=jnp.float32)
        ones = jnp.ones((MROW, BL), dtype=jnp.bfloat16)
        m = jax.lax.dot_general(ones, xb, _CONTRACT_1_1,
                                preferred_element_type=jnp.float32)

        @pl.when(i == 0)
        def _init():
            g_ref[...] = g
            m_ref[...] = m

        @pl.when(i != 0)
        def _acc():
            g_ref[...] += g
            m_ref[...] += m

    @pl.when((p == 1) & (i == 0))
    def _finalize():
        wt = w1t_ref[...]
        mu = m_ref[0:1, :] / N_ROWS
        mean = jnp.dot(mu, wt,
                       preferred_element_type=jnp.float32)
        t = jnp.dot(g_ref[...] / N_ROWS, wt,
                    preferred_element_type=jnp.float32)
        ex2 = jnp.sum(t * wt, axis=0, keepdims=True)
        var = ex2 - mean * mean
        scale = gamma_ref[...] * jax.lax.rsqrt(var + 1e-5)
        shift = beta_ref[...] - mean * scale
        w1s_ref[...] = (wt * scale).astype(jnp.bfloat16)
        sh_ref[...] = shift.T
        b2c_ref[...] = b2_ref[...].T
        w2h_ref[...] = (w2t_ref[...] * 0.5).astype(jnp.bfloat16)

    @pl.when(p == 1)
    def _apply():
        xb = x_ref[...].astype(jnp.bfloat16)
        h = jax.lax.dot_general(w1s_ref[...], xb, _CONTRACT_0_0,
                                preferred_element_type=jnp.float32)
        hn = (h + jnp.broadcast_to(sh_ref[...], (HID, BL))
              ).astype(jnp.bfloat16)
        e = jax.lax.erf(hn * jnp.bfloat16(1.0 / math.sqrt(2.0)))
        gl2 = hn + hn * e
        o = jnp.dot(w2h_ref[...], gl2,
                    preferred_element_type=jnp.float32)
        ob = o + jnp.broadcast_to(b2c_ref[...], (OUT, BL))
        out_ref[...] = jax.nn.sigmoid(ob)


def kernel(feat, W1, b1, gamma, beta, W2, b2):
    del b1
    outT = pl.pallas_call(
        _fused_kernel,
        grid=(2, NBL),
        in_specs=[
            pl.BlockSpec((IN_CH, BL), lambda p, i: (0, i)),
            pl.BlockSpec((IN_CH, HID), lambda p, i: (0, 0)),
            pl.BlockSpec((1, HID), lambda p, i: (0, 0)),
            pl.BlockSpec((1, HID), lambda p, i: (0, 0)),
            pl.BlockSpec((OUT, HID), lambda p, i: (0, 0)),
            pl.BlockSpec((1, OUT), lambda p, i: (0, 0)),
        ],
        out_specs=pl.BlockSpec((OUT, BL), lambda p, i: (0, i)),
        out_shape=jax.ShapeDtypeStruct((OUT, N_ROWS), jnp.float32),
        scratch_shapes=[
            pltpu.VMEM((IN_CH, IN_CH), jnp.float32),
            pltpu.VMEM((MROW, IN_CH), jnp.float32),
            pltpu.VMEM((IN_CH, HID), jnp.bfloat16),
            pltpu.VMEM((HID, 1), jnp.float32),
            pltpu.VMEM((OUT, 1), jnp.float32),
            pltpu.VMEM((OUT, HID), jnp.bfloat16),
        ],
        compiler_params=pltpu.CompilerParams(
            dimension_semantics=("arbitrary", "arbitrary")),
    )(feat.T, W1.T, gamma[None, :], beta[None, :], W2, b2[None, :])

    return outT.T


# speedup vs baseline: 8.4722x; 1.0704x over previous
import math

import jax
import jax.numpy as jnp
from jax.experimental import pallas as pl
from jax.experimental.pallas import tpu as pltpu

N_ROWS = 200000
BL = 16384
NBL = -(-N_ROWS // BL)
IN_CH = 32
HID = 64
OUT = 4
MROW = 8

_CONTRACT_0_0 = (((0,), (0,)), ((), ()))
_CONTRACT_1_1 = (((1,), (1,)), ((), ()))


def _fused_kernel(x_ref, w1t_ref, gamma_ref, beta_ref, w2t_ref, b2_ref,
                  out_ref, g_ref, m_ref, w1s_ref, sh_ref, b2c_ref, w2h_ref,
                  xc_ref):
    p = pl.program_id(0)
    i = pl.program_id(1)

    @pl.when(p == 0)
    def _stats():
        x = x_ref[...]
        lane = jax.lax.broadcasted_iota(jnp.int32, (IN_CH, BL), 1)
        x = jnp.where(lane < (N_ROWS - i * BL), x, 0.0)
        xb = x.astype(jnp.bfloat16)
        xc_ref[:, pl.ds(i * BL, BL)] = xb
        g = jax.lax.dot_general(xb, xb, _CONTRACT_1_1,
                                preferred_element_type=jnp.float32)
        ones = jnp.ones((MROW, BL), dtype=jnp.bfloat16)
        m = jax.lax.dot_general(ones, xb, _CONTRACT_1_1,
                                preferred_element_type=jnp.float32)

        @pl.when(i == 0)
        def _init():
            g_ref[...] = g
            m_ref[...] = m

        @pl.when(i != 0)
        def _acc():
            g_ref[...] += g
            m_ref[...] += m

    @pl.when((p == 1) & (i == 0))
    def _finalize():
        wt = w1t_ref[...]
        mu = m_ref[0:1, :] / N_ROWS
        mean = jnp.dot(mu, wt,
                       preferred_element_type=jnp.float32)
        t = jnp.dot(g_ref[...] / N_ROWS, wt,
                    preferred_element_type=jnp.float32)
        ex2 = jnp.sum(t * wt, axis=0, keepdims=True)
        var = ex2 - mean * mean
        scale = gamma_ref[...] * jax.lax.rsqrt(var + 1e-5)
        shift = beta_ref[...] - mean * scale
        w1s_ref[...] = (wt * scale).astype(jnp.bfloat16)
        sh_ref[...] = shift.T
        b2c_ref[...] = b2_ref[...].T
        w2h_ref[...] = (w2t_ref[...] * 0.5).astype(jnp.bfloat16)

    @pl.when(p == 1)
    def _apply():
        xb = xc_ref[:, pl.ds(i * BL, BL)]
        h = jax.lax.dot_general(w1s_ref[...], xb, _CONTRACT_0_0,
                                preferred_element_type=jnp.float32)
        hn = (h + jnp.broadcast_to(sh_ref[...], (HID, BL))
              ).astype(jnp.bfloat16)
        e = jax.lax.erf(hn * jnp.bfloat16(1.0 / math.sqrt(2.0)))
        gl2 = hn + hn * e
        o = jnp.dot(w2h_ref[...], gl2,
                    preferred_element_type=jnp.float32)
        ob = o + jnp.broadcast_to(b2c_ref[...], (OUT, BL))
        out_ref[...] = jax.nn.sigmoid(ob)


def kernel(feat, W1, b1, gamma, beta, W2, b2):
    del b1
    outT = pl.pallas_call(
        _fused_kernel,
        grid=(2, NBL),
        in_specs=[
            pl.BlockSpec((IN_CH, BL), lambda p, i: (0, i * (1 - p))),
            pl.BlockSpec((IN_CH, HID), lambda p, i: (0, 0)),
            pl.BlockSpec((1, HID), lambda p, i: (0, 0)),
            pl.BlockSpec((1, HID), lambda p, i: (0, 0)),
            pl.BlockSpec((OUT, HID), lambda p, i: (0, 0)),
            pl.BlockSpec((1, OUT), lambda p, i: (0, 0)),
        ],
        out_specs=pl.BlockSpec((OUT, BL), lambda p, i: (0, i)),
        out_shape=jax.ShapeDtypeStruct((OUT, N_ROWS), jnp.float32),
        scratch_shapes=[
            pltpu.VMEM((IN_CH, IN_CH), jnp.float32),
            pltpu.VMEM((MROW, IN_CH), jnp.float32),
            pltpu.VMEM((IN_CH, HID), jnp.bfloat16),
            pltpu.VMEM((HID, 1), jnp.float32),
            pltpu.VMEM((OUT, 1), jnp.float32),
            pltpu.VMEM((OUT, HID), jnp.bfloat16),
            pltpu.VMEM((IN_CH, NBL * BL), jnp.bfloat16),
        ],
        compiler_params=pltpu.CompilerParams(
            dimension_semantics=("arbitrary", "arbitrary")),
    )(feat.T, W1.T, gamma[None, :], beta[None, :], W2, b2[None, :])

    return outT.T
